# Initial kernel scaffold; baseline (speedup 1.0000x reference)
import jax
import jax.numpy as jnp
from jax.experimental import pallas as pl


def kernel(output, mask, ind, rotbin, rotres):
    raise NotImplementedError("write your pallas kernel here")


# speedup vs baseline: 3.4786x; 3.4786x over previous
import functools

import jax
import jax.numpy as jnp
from jax import lax
from jax.experimental import pallas as pl
from jax.experimental.pallas import tpu as pltpu
from jax.experimental.pallas import tpu_sc as plsc

B, C, H, W, K = 64, 8, 128, 128, 128
HW = H * W
N = B * K
NC, NS = 2, 16
NW = NC * NS
IPT = N // NW
GPT = C * IPT


def _sc_gather_body(src_hbm, ind_hbm, out_hbm, ind_v, idx_v, rows_v, sem):
    wid = lax.axis_index("s") * NC + lax.axis_index("c")
    base_item = wid * IPT
    pltpu.sync_copy(ind_hbm.at[pl.ds(base_item, IPT)], ind_v)
    wbase = (wid * (IPT // K)) * (C * HW)
    for c in range(C):
        for chunk in range(IPT // 16):
            off = (chunk // (K // 16)) * (C * HW) + c * HW
            idx_v[pl.ds((c * (IPT // 16) + chunk) * 16, 16)] = (
                ind_v[pl.ds(chunk * 16, 16)] + (wbase + off)
            )
    pltpu.async_copy(src_hbm.at[idx_v], rows_v, sem).wait()
    for c in range(C):
        pltpu.sync_copy(
            rows_v.at[pl.ds(c * IPT, IPT)],
            out_hbm.at[c, pl.ds(base_item, IPT)],
        )


@functools.cache
def _sc_gather():
    return functools.partial(
        pl.kernel,
        out_type=jax.ShapeDtypeStruct((C, N), jnp.float32),
        mesh=plsc.VectorSubcoreMesh(core_axis_name="c", subcore_axis_name="s"),
        scratch_types=[
            pltpu.VMEM((IPT,), jnp.int32),
            pltpu.VMEM((GPT,), jnp.int32),
            pltpu.VMEM((GPT,), jnp.float32),
            pltpu.SemaphoreType.DMA,
        ],
    )(_sc_gather_body)


def _loss_body(pred_ref, mask_ref, tb_ref, tr_ref, out_ref):
    x = [pred_ref[c] for c in range(C)]
    mf = mask_ref[...].astype(jnp.float32)
    tb0 = tb_ref[0]
    tb1 = tb_ref[1]
    tr0 = tr_ref[0]
    tr1 = tr_ref[1]
    cnt = jnp.sum(mf)

    def pick_logp(a, b, t):
        m = jnp.maximum(a, b)
        lse = m + jnp.log(jnp.exp(a - m) + jnp.exp(b - m))
        return jnp.where(t == 1, b, a) - lse

    s1 = jnp.sum(pick_logp(x[0], x[1], tb0) * mf)
    s2 = jnp.sum(pick_logp(x[4], x[5], tb1) * mf)

    def sl1(p, t):
        d = jnp.abs(p - t)
        return jnp.where(d < 1.0, 0.5 * d * d, d - 0.5)

    w1 = tb0.astype(jnp.float32)
    w2 = tb1.astype(jnp.float32)
    n1 = jnp.sum(w1)
    n2 = jnp.sum(w2)
    r1 = jnp.sum((sl1(x[2], jnp.sin(tr0)) + sl1(x[3], jnp.cos(tr0))) * w1)
    r2 = jnp.sum((sl1(x[6], jnp.sin(tr1)) + sl1(x[7], jnp.cos(tr1))) * w2)

    zero = jnp.float32(0.0)
    lb1 = jnp.where(cnt > 0, -s1 / cnt, zero)
    lb2 = jnp.where(cnt > 0, -s2 / cnt, zero)
    lr = jnp.where(n1 > 0, r1 / n1, zero) + jnp.where(n2 > 0, r2 / n2, zero)
    total = lb1 + lb2 + lr
    out_ref[0, 0] = jnp.where(cnt == 0, zero, total)


_loss = pl.pallas_call(
    _loss_body,
    out_shape=jax.ShapeDtypeStruct((1, 1), jnp.float32),
    out_specs=pl.BlockSpec(memory_space=pltpu.SMEM),
)


def kernel(output, mask, ind, rotbin, rotres):
    src = output.reshape(-1)
    indf = ind.reshape(-1).astype(jnp.int32)
    pred = _sc_gather()(src, indf)
    pred3 = pred.reshape(C, B, K)
    tb = jnp.transpose(rotbin, (2, 0, 1)).astype(jnp.int32)
    tr = jnp.transpose(rotres, (2, 0, 1))
    out = _loss(pred3, mask.astype(jnp.int32), tb, tr)
    return out[0, 0]
